# Initial kernel scaffold; baseline (speedup 1.0000x reference)
import jax
import jax.numpy as jnp
from jax.experimental import pallas as pl


def kernel(x, params):
    raise NotImplementedError("write your pallas kernel here")


# speedup vs baseline: 5.7681x; 5.7681x over previous
import jax
import jax.numpy as jnp
from jax import lax
from jax.experimental import pallas as pl

B = 8
C = 160
LAST = 256
N = 512
K = 16
NBLK = 4
CH1 = 256

_F32 = jnp.float32
_BF16 = jnp.bfloat16
_INF = float('inf')


def _dot(a, b, dims):
    return lax.dot_general(a.astype(_BF16), b.astype(_BF16), (dims, ((), ())),
                           preferred_element_type=_F32)


def _split3(x):
    t1 = x.astype(_BF16)
    r1 = x - t1.astype(_F32)
    t2 = r1.astype(_BF16)
    t3 = (r1 - t2.astype(_F32)).astype(_BF16)
    return t1, t2, t3


def _tc_body(xrT_ref, WcdT_ref, bcd_ref, Wg_ref, bg_ref, sg_ref, bb_ref,
             W1_ref, b1_ref, s1_ref, be1_ref, W2_ref, b2_ref, s2_ref, be2_ref,
             Wc1T_ref, bc1_ref, Wc2T_ref, bc2_ref, out_ref):
    X = _dot(WcdT_ref[...], xrT_ref[0], ((1,), (0,))) + bcd_ref[...]

    iota = lax.broadcasted_iota(jnp.int32, (N, N), 1)

    for i in range(NBLK):
        xs_cm = X.T
        nrm = jnp.sqrt(jnp.sum(xs_cm * xs_cm, axis=0, keepdims=True))
        v = xs_cm / (nrm + 1e-12)
        sq = jnp.sum(v * v, axis=0)
        vt = v.T
        G = _dot(vt, vt, ((1,), (1,)))
        d = (sq[:, None] + sq[None, :]) - 2.0 * G

        X1, X2, X3 = _split3(X)
        Wg = Wg_ref[i].astype(_BF16)
        bg = bg_ref[i]
        sg = sg_ref[i]
        bb = bb_ref[i]

        def step(_, carry):
            Dm, Y = carry
            cur = jnp.min(Dm, axis=1, keepdims=True)
            amin = jnp.min(jnp.where(Dm == cur, iota, N), axis=1)
            oh = iota == amin[:, None]
            ohb = oh.astype(_BF16)
            xj = (lax.dot_general(ohb, X1, (((1,), (0,)), ((), ())),
                                  preferred_element_type=_F32)
                  + lax.dot_general(ohb, X2, (((1,), (0,)), ((), ())),
                                    preferred_element_type=_F32)) \
                 + lax.dot_general(ohb, X3, (((1,), (0,)), ((), ())),
                                   preferred_element_type=_F32)
            feat = jnp.concatenate([X, xj - X], axis=1)
            yk = lax.dot_general(feat.astype(_BF16), Wg, (((1,), (0,)), ((), ())),
                                 preferred_element_type=_F32)
            yk = (yk + bg) * sg + bb
            Y = jnp.maximum(Y, jnp.maximum(yk, 0.0))
            Dm = jnp.where(oh, _INF, Dm)
            return Dm, Y

        _, Y = lax.fori_loop(0, K, step,
                             (d, jnp.full((N, C), -_INF, dtype=_F32)))

        Xe = X + Y
        H = (_dot(Xe, W1_ref[i], ((1,), (0,))) + b1_ref[i]) * s1_ref[i] + be1_ref[i]
        H = jnp.maximum(H, 0.0)
        X = (_dot(H, W2_ref[i], ((1,), (0,))) + b2_ref[i]) * s2_ref[i] + be2_ref[i]

    h1 = _dot(Wc1T_ref[...], X, ((1,), (0,))) + bc1_ref[...]
    h1 = jnp.maximum(h1, 0.0)
    o = _dot(Wc2T_ref[...], h1, ((1,), (0,))) + bc2_ref[...]
    out_ref[0] = jnp.maximum(o, 0.0)


def _stack_params(params):
    inv = 1.0 / jnp.sqrt(1.0 + 1e-5)
    get = lambda n: jnp.stack([params['b%d_%s' % (i, n)] for i in range(NBLK)])
    Wg = get('Wg')
    W1 = get('W1')
    W2 = get('W2')
    row = lambda a: a[:, None, :]
    return (Wg, row(get('bg')), row(get('gg') * inv), row(get('bb')),
            W1, row(get('b1')), row(get('g1') * inv), row(get('be1')),
            W2, row(get('b2')), row(get('g2') * inv), row(get('be2')))


def kernel(x, params):
    xrT = x.reshape(B, LAST, C)
    sp = _stack_params(params)
    args = (xrT, params['W_cd'].T, params['b_cd'][:, None]) + sp + (
        params['Wc1'].T, params['bc1'][:, None],
        params['Wc2'].T, params['bc2'][:, None])

    bspec = lambda shp: pl.BlockSpec(shp, lambda b: (0,) * len(shp))
    in_specs = [pl.BlockSpec((1, LAST, C), lambda b: (b, 0, 0))]
    in_specs += [bspec(a.shape) for a in args[1:]]
    out = pl.pallas_call(
        _tc_body,
        grid=(B,),
        in_specs=in_specs,
        out_specs=pl.BlockSpec((1, 1, C), lambda b: (b, 0, 0)),
        out_shape=jax.ShapeDtypeStruct((B, 1, C), _F32),
    )(*args)
    return out.reshape(B, C)
